# Initial kernel scaffold; baseline (speedup 1.0000x reference)
import jax
import jax.numpy as jnp
from jax.experimental import pallas as pl


def kernel(x, sigma2):
    raise NotImplementedError("write your pallas kernel here")


# speedup vs baseline: 8.6037x; 8.6037x over previous
import jax
import jax.numpy as jnp
from jax.experimental import pallas as pl
from jax.experimental.pallas import tpu as pltpu

_BLOCK_ROWS = 256


def _diag_add_body(x_ref, s_ref, o_ref):
    i = pl.program_id(0)
    blk = x_ref[...]
    rows, cols = blk.shape
    r = jax.lax.broadcasted_iota(jnp.int32, (rows, cols), 0)
    c = jax.lax.broadcasted_iota(jnp.int32, (rows, cols), 1)
    mask = c == r + i * rows
    o_ref[...] = blk + jnp.where(mask, s_ref[0], jnp.float32(0.0))


def kernel(x, sigma2):
    n, m = x.shape
    br = _BLOCK_ROWS if n % _BLOCK_ROWS == 0 else n
    grid = (n // br,)
    return pl.pallas_call(
        _diag_add_body,
        grid=grid,
        in_specs=[
            pl.BlockSpec((br, m), lambda i: (i, 0)),
            pl.BlockSpec(memory_space=pltpu.SMEM),
        ],
        out_specs=pl.BlockSpec((br, m), lambda i: (i, 0)),
        out_shape=jax.ShapeDtypeStruct((n, m), x.dtype),
    )(x, sigma2)
